# baseline (speedup 1.0000x reference)
import jax
import jax.numpy as jnp
from jax.experimental import pallas as pl
from jax.experimental.pallas import tpu as pltpu

_ROWS, _COLS = 16384, 200
_BLK = 8192


def _mod_body(fi_ref, vs_ref, x_ref, o_ref):
    v = vs_ref[fi_ref[0]]
    rv = 1.0 / v.astype(jnp.float32)
    x = x_ref[...]
    q = jnp.floor(x.astype(jnp.float32) * rv).astype(jnp.int32)
    r = x - q * v
    r = jnp.where(r < 0, r + v, r)
    r = jnp.where(r >= v, r - v, r)
    o_ref[...] = r


def kernel(indices, feature_idx, vocab_sizes):
    fi = jnp.reshape(jnp.asarray(feature_idx, dtype=jnp.int32), (1,))
    xt = jnp.swapaxes(indices, 0, 1)
    grid = (_ROWS // _BLK,)
    out_t = pl.pallas_call(
        _mod_body,
        grid=grid,
        in_specs=[
            pl.BlockSpec(memory_space=pltpu.SMEM),
            pl.BlockSpec(memory_space=pltpu.SMEM),
            pl.BlockSpec((_COLS, _BLK), lambda i: (0, i)),
        ],
        out_specs=pl.BlockSpec((_COLS, _BLK), lambda i: (0, i)),
        out_shape=jax.ShapeDtypeStruct((_COLS, _ROWS), indices.dtype),
    )(fi, vocab_sizes, xt)
    return jnp.swapaxes(out_t, 0, 1)


# speedup vs baseline: 1.0802x; 1.0802x over previous
import jax
import jax.numpy as jnp
from jax.experimental import pallas as pl
from jax.experimental.pallas import tpu as pltpu

_ROWS, _COLS = 16384, 200
_BLK = 4096


def _mod_body(fi_ref, vs_ref, x_ref, o_ref):
    v = vs_ref[fi_ref[0]]
    rv = 1.0 / v.astype(jnp.float32)
    x = x_ref[...]
    q = jnp.floor(x.astype(jnp.float32) * rv).astype(jnp.int32)
    r = x - q * v
    r = jnp.where(r < 0, r + v, r)
    r = jnp.where(r >= v, r - v, r)
    o_ref[...] = r


def kernel(indices, feature_idx, vocab_sizes):
    fi = jnp.reshape(jnp.asarray(feature_idx, dtype=jnp.int32), (1,))
    xt = jnp.swapaxes(indices, 0, 1)
    grid = (_ROWS // _BLK,)
    out_t = pl.pallas_call(
        _mod_body,
        grid=grid,
        in_specs=[
            pl.BlockSpec(memory_space=pltpu.SMEM),
            pl.BlockSpec(memory_space=pltpu.SMEM),
            pl.BlockSpec((_COLS, _BLK), lambda i: (0, i)),
        ],
        out_specs=pl.BlockSpec((_COLS, _BLK), lambda i: (0, i)),
        out_shape=jax.ShapeDtypeStruct((_COLS, _ROWS), indices.dtype),
    )(fi, vocab_sizes, xt)
    return jnp.swapaxes(out_t, 0, 1)
